# Initial kernel scaffold; baseline (speedup 1.0000x reference)
import jax
import jax.numpy as jnp
from jax.experimental import pallas as pl


def kernel(hidden_states, gate_w, w1, w3, w2):
    raise NotImplementedError("write your pallas kernel here")


# speedup vs baseline: 2.5970x; 2.5970x over previous
import functools

import jax
import jax.numpy as jnp
from jax import lax
from jax.experimental import pallas as pl
from jax.experimental.pallas import tpu as pltpu
from jax.experimental.pallas import tpu_sc as plsc

_E = 64
_K = 2
_D = 1024
_F = 1024
_T = 2048
_A = _T * _K
_BLK = 128
_NT = _A // _BLK + _E
_PAD = _NT * _BLK
_CH = 64


def _router_body(x_ref, gw_ref, e0_ref, e1_ref, p0_ref, p1_ref):
    lt = lax.dot_general(gw_ref[...], x_ref[...], (((1,), (1,)), ((), ())),
                         preferred_element_type=jnp.float32)
    a1 = jnp.argmax(lt, axis=0).astype(jnp.int32)
    m1 = jnp.max(lt, axis=0)
    ii = lax.broadcasted_iota(jnp.int32, lt.shape, 0)
    lt2 = jnp.where(ii == a1[None, :], -jnp.inf, lt)
    a2 = jnp.argmax(lt2, axis=0).astype(jnp.int32)
    m2 = jnp.max(lt2, axis=0)
    p = jax.nn.sigmoid(m1 - m2)
    e0_ref[...] = jnp.broadcast_to(a1[None, :], e0_ref.shape)
    e1_ref[...] = jnp.broadcast_to(a2[None, :], e1_ref.shape)
    p0_ref[...] = jnp.broadcast_to(p[None, :], p0_ref.shape)
    p1_ref[...] = jnp.broadcast_to((1.0 - p)[None, :], p1_ref.shape)


def _route(x, gate_w):
    outs = pl.pallas_call(
        _router_body,
        out_shape=[
            jax.ShapeDtypeStruct((8, _T), jnp.int32),
            jax.ShapeDtypeStruct((8, _T), jnp.int32),
            jax.ShapeDtypeStruct((8, _T), jnp.float32),
            jax.ShapeDtypeStruct((8, _T), jnp.float32),
        ],
    )(x, gate_w)
    return outs[0][0], outs[1][0], outs[2][0], outs[3][0]


def _route_meta(e0, e1, p0, p1):
    ef = jnp.stack([e0, e1], axis=1).reshape(-1)
    wf = jnp.stack([p0, p1], axis=1).reshape(-1)
    oh = (ef[:, None] == jnp.arange(_E, dtype=jnp.int32)[None, :]).astype(jnp.int32)
    cs = jnp.cumsum(oh, axis=0)
    rank = jnp.take_along_axis(cs, ef[:, None], axis=1)[:, 0] - 1
    counts = cs[-1]
    tiles = (counts + _BLK - 1) // _BLK
    ends = jnp.cumsum(tiles).astype(jnp.int32)
    starts = ends - tiles
    ntile = ends[-1]
    pos = starts[ef] * _BLK + rank
    rows = jnp.zeros((_PAD,), jnp.int32).at[pos].set(
        jnp.arange(_A, dtype=jnp.int32) // _K)
    wsrt = jnp.zeros((_PAD,), jnp.float32).at[pos].set(wf)
    tt = jnp.arange(_NT, dtype=jnp.int32)
    texp = jnp.searchsorted(ends, jnp.minimum(tt, ntile - 1),
                            side="right").astype(jnp.int32)
    return rows, wsrt, texp, ntile.reshape(1), pos


_NC = 2
_NS = 16
_NW = _NC * _NS


@functools.lru_cache(maxsize=None)
def _make_sc_gather(n_idx, d):
    per_w = n_idx // _NW
    nch = per_w // _CH
    assert per_w % _CH == 0 and n_idx % _NW == 0

    mesh = plsc.VectorSubcoreMesh(core_axis_name="c", subcore_axis_name="s",
                                  num_cores=_NC, num_subcores=_NS)

    @functools.partial(
        pl.kernel,
        mesh=mesh,
        out_type=jax.ShapeDtypeStruct((n_idx, d), jnp.float32),
        scratch_types=[
            pltpu.VMEM((_CH,), jnp.int32),
            pltpu.VMEM((_CH, d), jnp.float32),
            pltpu.SemaphoreType.DMA,
        ],
    )
    def g(table_hbm, idx_hbm, out_hbm, idx_v, buf_v, sem):
        wid = lax.axis_index("s") * _NC + lax.axis_index("c")
        base = wid * per_w

        def chunk(i, carry):
            off = base + i * _CH
            pltpu.sync_copy(idx_hbm.at[pl.ds(off, _CH)], idx_v)
            pltpu.async_copy(table_hbm.at[idx_v], buf_v, sem).wait()
            pltpu.sync_copy(buf_v, out_hbm.at[pl.ds(off, _CH)])
            return carry

        lax.fori_loop(0, nch, chunk, 0)

    return g


def _ffn_body(texp_ref, nt_ref, x_ref, w1_ref, w3_ref, w2_ref, ws_ref, y_ref):
    t = pl.program_id(0)

    @pl.when(t < nt_ref[0])
    def _():
        x = x_ref[...]
        h1 = jnp.dot(x, w1_ref[0], preferred_element_type=jnp.float32)
        h3 = jnp.dot(x, w3_ref[0], preferred_element_type=jnp.float32)
        h = (h1 * jax.nn.sigmoid(h1)) * h3
        y = jnp.dot(h, w2_ref[0], preferred_element_type=jnp.float32)
        y_ref[...] = y * ws_ref[0, 0][:, None]


def _ffn(texp, nt, xs, w1, w3, w2, wsrt3):
    grid_spec = pltpu.PrefetchScalarGridSpec(
        num_scalar_prefetch=2,
        grid=(_NT,),
        in_specs=[
            pl.BlockSpec((_BLK, _D),
                         lambda t, texp, nt: (jnp.minimum(t, nt[0] - 1), 0)),
            pl.BlockSpec((1, _D, _F), lambda t, texp, nt: (texp[t], 0, 0)),
            pl.BlockSpec((1, _D, _F), lambda t, texp, nt: (texp[t], 0, 0)),
            pl.BlockSpec((1, _F, _D), lambda t, texp, nt: (texp[t], 0, 0)),
            pl.BlockSpec((1, 1, _BLK),
                         lambda t, texp, nt: (jnp.minimum(t, nt[0] - 1), 0, 0)),
        ],
        out_specs=pl.BlockSpec((_BLK, _D), lambda t, texp, nt: (t, 0)),
    )
    return pl.pallas_call(
        _ffn_body,
        grid_spec=grid_spec,
        out_shape=jax.ShapeDtypeStruct((_PAD, _D), jnp.float32),
    )(texp, nt, xs, w1, w3, w2, wsrt3)


def _add_body(z_ref, o_ref):
    z = z_ref[...]
    o_ref[...] = z[:, 0, :] + z[:, 1, :]


def _pair_add(z):
    return pl.pallas_call(
        _add_body,
        grid=(_T // _BLK,),
        in_specs=[pl.BlockSpec((_BLK, _K, _D), lambda t: (t, 0, 0))],
        out_specs=pl.BlockSpec((_BLK, _D), lambda t: (t, 0)),
        out_shape=jax.ShapeDtypeStruct((_T, _D), jnp.float32),
    )(z)


def kernel(hidden_states, gate_w, w1, w3, w2):
    orig_shape = hidden_states.shape
    x = hidden_states.reshape(_T, _D)
    e0, e1, p0, p1 = _route(x, gate_w)
    rows, wsrt, texp, nt, pos = _route_meta(e0, e1, p0, p1)
    xs = _make_sc_gather(_PAD, _D)(x, rows)
    y = _ffn(texp, nt, xs, w1, w3, w2, wsrt.reshape(_NT, 1, _BLK))
    z = _make_sc_gather(_A, _D)(y, pos)
    out = _pair_add(z.reshape(_T, _K, _D))
    return out.reshape(orig_shape)


# speedup vs baseline: 4.5635x; 1.7572x over previous
import functools

import jax
import jax.numpy as jnp
from jax import lax
from jax.experimental import pallas as pl
from jax.experimental.pallas import tpu as pltpu
from jax.experimental.pallas import tpu_sc as plsc

_E = 64
_K = 2
_D = 1024
_F = 1024
_T = 2048
_A = _T * _K
_BLK = 128
_NT = _A // _BLK + _E
_PAD = _NT * _BLK
_CH = 64


def _router_body(x_ref, gw_ref, e0_ref, e1_ref, p0_ref, p1_ref):
    lt = lax.dot_general(gw_ref[...], x_ref[...], (((1,), (1,)), ((), ())),
                         preferred_element_type=jnp.float32)
    a1 = jnp.argmax(lt, axis=0).astype(jnp.int32)
    m1 = jnp.max(lt, axis=0)
    ii = lax.broadcasted_iota(jnp.int32, lt.shape, 0)
    lt2 = jnp.where(ii == a1[None, :], -jnp.inf, lt)
    a2 = jnp.argmax(lt2, axis=0).astype(jnp.int32)
    m2 = jnp.max(lt2, axis=0)
    p = jax.nn.sigmoid(m1 - m2)
    e0_ref[...] = jnp.broadcast_to(a1[None, :], e0_ref.shape)
    e1_ref[...] = jnp.broadcast_to(a2[None, :], e1_ref.shape)
    p0_ref[...] = jnp.broadcast_to(p[None, :], p0_ref.shape)
    p1_ref[...] = jnp.broadcast_to((1.0 - p)[None, :], p1_ref.shape)


def _route(x, gate_w):
    outs = pl.pallas_call(
        _router_body,
        out_shape=[
            jax.ShapeDtypeStruct((8, _T), jnp.int32),
            jax.ShapeDtypeStruct((8, _T), jnp.int32),
            jax.ShapeDtypeStruct((8, _T), jnp.float32),
            jax.ShapeDtypeStruct((8, _T), jnp.float32),
        ],
    )(x, gate_w)
    return outs[0][0], outs[1][0], outs[2][0], outs[3][0]


def _route_meta(e0, e1, p0, p1):
    ef = jnp.stack([e0, e1], axis=1).reshape(-1)
    wf = jnp.stack([p0, p1], axis=1).reshape(-1)
    oh = (ef[:, None] == jnp.arange(_E, dtype=jnp.int32)[None, :]).astype(jnp.int32)
    cs = jnp.cumsum(oh, axis=0)
    rank = jnp.take_along_axis(cs, ef[:, None], axis=1)[:, 0] - 1
    counts = cs[-1]
    tiles = (counts + _BLK - 1) // _BLK
    ends = jnp.cumsum(tiles).astype(jnp.int32)
    starts = ends - tiles
    ntile = ends[-1]
    pos = starts[ef] * _BLK + rank
    rows = (jnp.arange(_PAD, dtype=jnp.int32) % _T).at[pos].set(
        jnp.arange(_A, dtype=jnp.int32) // _K)
    wsrt = jnp.zeros((_PAD,), jnp.float32).at[pos].set(wf)
    tt = jnp.arange(_NT, dtype=jnp.int32)
    texp = jnp.searchsorted(ends, jnp.minimum(tt, ntile - 1),
                            side="right").astype(jnp.int32)
    return rows, wsrt, texp, ntile.reshape(1), pos


_NC = 2
_NS = 16
_NW = _NC * _NS


@functools.lru_cache(maxsize=None)
def _make_sc_gather(n_idx, d):
    per_w = n_idx // _NW
    nch = per_w // _CH
    assert per_w % _CH == 0 and n_idx % _NW == 0

    mesh = plsc.VectorSubcoreMesh(core_axis_name="c", subcore_axis_name="s",
                                  num_cores=_NC, num_subcores=_NS)

    @functools.partial(
        pl.kernel,
        mesh=mesh,
        out_type=jax.ShapeDtypeStruct((n_idx, d), jnp.float32),
        scratch_types=[
            pltpu.VMEM((_CH,), jnp.int32),
            pltpu.VMEM((_CH, d), jnp.float32),
            pltpu.SemaphoreType.DMA,
        ],
    )
    def g(table_hbm, idx_hbm, out_hbm, idx_v, buf_v, sem):
        wid = lax.axis_index("s") * _NC + lax.axis_index("c")
        base = wid * per_w

        def chunk(i, carry):
            off = base + i * _CH
            pltpu.sync_copy(idx_hbm.at[pl.ds(off, _CH)], idx_v)
            pltpu.async_copy(table_hbm.at[idx_v], buf_v, sem).wait()
            pltpu.sync_copy(buf_v, out_hbm.at[pl.ds(off, _CH)])
            return carry

        lax.fori_loop(0, nch, chunk, 0)

    return g


def _ffn_body(texp_ref, nt_ref, x_ref, w1_ref, w3_ref, w2_ref, ws_ref, y_ref):
    t = pl.program_id(0)

    @pl.when(t < nt_ref[0])
    def _():
        x = x_ref[...]
        h1 = jnp.dot(x, w1_ref[0], preferred_element_type=jnp.float32)
        h3 = jnp.dot(x, w3_ref[0], preferred_element_type=jnp.float32)
        h = (h1 * jax.nn.sigmoid(h1)) * h3
        y = jnp.dot(h, w2_ref[0], preferred_element_type=jnp.float32)
        y_ref[...] = y * ws_ref[0, 0][:, None]


def _ffn(texp, nt, xs, w1, w3, w2, wsrt3):
    grid_spec = pltpu.PrefetchScalarGridSpec(
        num_scalar_prefetch=2,
        grid=(_NT,),
        in_specs=[
            pl.BlockSpec((_BLK, _D),
                         lambda t, texp, nt: (jnp.minimum(t, nt[0] - 1), 0)),
            pl.BlockSpec((1, _D, _F), lambda t, texp, nt: (texp[t], 0, 0)),
            pl.BlockSpec((1, _D, _F), lambda t, texp, nt: (texp[t], 0, 0)),
            pl.BlockSpec((1, _F, _D), lambda t, texp, nt: (texp[t], 0, 0)),
            pl.BlockSpec((1, 1, _BLK),
                         lambda t, texp, nt: (jnp.minimum(t, nt[0] - 1), 0, 0)),
        ],
        out_specs=pl.BlockSpec((_BLK, _D), lambda t, texp, nt: (t, 0)),
    )
    return pl.pallas_call(
        _ffn_body,
        grid_spec=grid_spec,
        out_shape=jax.ShapeDtypeStruct((_PAD, _D), jnp.float32),
    )(texp, nt, xs, w1, w3, w2, wsrt3)


def _add_body(z_ref, o_ref):
    z = z_ref[...]
    o_ref[...] = z[:, 0, :] + z[:, 1, :]


def _pair_add(z):
    return pl.pallas_call(
        _add_body,
        grid=(_T // _BLK,),
        in_specs=[pl.BlockSpec((_BLK, _K, _D), lambda t: (t, 0, 0))],
        out_specs=pl.BlockSpec((_BLK, _D), lambda t: (t, 0)),
        out_shape=jax.ShapeDtypeStruct((_T, _D), jnp.float32),
    )(z)


def kernel(hidden_states, gate_w, w1, w3, w2):
    orig_shape = hidden_states.shape
    x = hidden_states.reshape(_T, _D)
    e0, e1, p0, p1 = _route(x, gate_w)
    rows, wsrt, texp, nt, pos = _route_meta(e0, e1, p0, p1)
    xs = _make_sc_gather(_PAD, _D)(x, rows)
    y = _ffn(texp, nt, xs, w1, w3, w2, wsrt.reshape(_NT, 1, _BLK))
    z = _make_sc_gather(_A, _D)(y, pos)
    out = _pair_add(z.reshape(_T, _K, _D))
    return out.reshape(orig_shape)


# speedup vs baseline: 6.0588x; 1.3277x over previous
import functools

import jax
import jax.numpy as jnp
from jax import lax
from jax.experimental import pallas as pl
from jax.experimental.pallas import tpu as pltpu
from jax.experimental.pallas import tpu_sc as plsc

_E = 64
_K = 2
_D = 1024
_F = 1024
_T = 2048
_A = _T * _K
_BLK = 128
_NT = _A // _BLK + _E
_PAD = _NT * _BLK
_CH = 64
_PC = 512


def _router_body(x_ref, gw_ref, e0_ref, e1_ref, p0_ref, p1_ref):
    lt = lax.dot_general(gw_ref[...], x_ref[...], (((1,), (1,)), ((), ())),
                         preferred_element_type=jnp.float32)
    a1 = jnp.argmax(lt, axis=0).astype(jnp.int32)
    m1 = jnp.max(lt, axis=0)
    ii = lax.broadcasted_iota(jnp.int32, lt.shape, 0)
    lt2 = jnp.where(ii == a1[None, :], -jnp.inf, lt)
    a2 = jnp.argmax(lt2, axis=0).astype(jnp.int32)
    m2 = jnp.max(lt2, axis=0)
    p = jax.nn.sigmoid(m1 - m2)
    e0_ref[...] = jnp.broadcast_to(a1[None, :], e0_ref.shape)
    e1_ref[...] = jnp.broadcast_to(a2[None, :], e1_ref.shape)
    p0_ref[...] = jnp.broadcast_to(p[None, :], p0_ref.shape)
    p1_ref[...] = jnp.broadcast_to((1.0 - p)[None, :], p1_ref.shape)


def _route(x, gate_w):
    return pl.pallas_call(
        _router_body,
        out_shape=[
            jax.ShapeDtypeStruct((8, _T), jnp.int32),
            jax.ShapeDtypeStruct((8, _T), jnp.int32),
            jax.ShapeDtypeStruct((8, _T), jnp.float32),
            jax.ShapeDtypeStruct((8, _T), jnp.float32),
        ],
    )(x, gate_w)


def _route_meta(e0, e1):
    ii = jnp.arange(_E, dtype=jnp.int32)[None, :]
    oh0 = (e0[:, None] == ii).astype(jnp.float32)
    oh1 = (e1[:, None] == ii).astype(jnp.float32)
    ohb = oh0 + oh1
    ci = jnp.arange(_PC, dtype=jnp.int32)
    tril = (ci[:, None] > ci[None, :]).astype(jnp.float32)
    ohc = ohb.reshape(_T // _PC, _PC, _E)
    within = jnp.einsum("ij,cjk->cik", tril, ohc,
                        preferred_element_type=jnp.float32)
    chunk_tot = jnp.cumsum(jnp.sum(ohc, axis=1), axis=0)
    carry = jnp.concatenate(
        [jnp.zeros((1, _E), jnp.float32), chunk_tot[:-1]], axis=0)
    prefix = (within + carry[:, None, :]).reshape(_T, _E)
    counts = chunk_tot[-1]
    tiles = jnp.ceil(counts / _BLK)
    ends = jnp.cumsum(tiles)
    starts = ends - tiles
    ntile = ends[-1].astype(jnp.int32)
    base = prefix + starts[None, :] * _BLK
    pos0 = jnp.sum(base * oh0, axis=1).astype(jnp.int32)
    pos1 = jnp.sum(base * oh1, axis=1).astype(jnp.int32)
    tt = jnp.arange(_NT, dtype=jnp.float32)
    tcl = jnp.minimum(tt, ends[-1] - 1.0)
    texp = jnp.sum((ends[None, :] <= tcl[:, None]).astype(jnp.int32),
                   axis=1).astype(jnp.int32)
    return pos0, pos1, texp, ntile.reshape(1)


_NC = 2
_NS = 16
_NW = _NC * _NS


def _sc_mesh():
    return plsc.VectorSubcoreMesh(core_axis_name="c", subcore_axis_name="s",
                                  num_cores=_NC, num_subcores=_NS)


@functools.lru_cache(maxsize=None)
def _make_sc_dispatch():
    per_w = _T // _NW

    @functools.partial(
        pl.kernel,
        mesh=_sc_mesh(),
        out_type=jax.ShapeDtypeStruct((_PAD, _D), jnp.float32),
        scratch_types=[
            pltpu.VMEM((per_w,), jnp.int32),
            pltpu.VMEM((per_w,), jnp.int32),
            pltpu.VMEM((per_w, _D), jnp.float32),
            pltpu.SemaphoreType.DMA,
        ],
    )
    def d(x_hbm, p0_hbm, p1_hbm, xs_hbm, i0_v, i1_v, buf_v, sem):
        wid = lax.axis_index("s") * _NC + lax.axis_index("c")
        base = wid * per_w
        pltpu.sync_copy(p0_hbm.at[pl.ds(base, per_w)], i0_v)
        pltpu.sync_copy(p1_hbm.at[pl.ds(base, per_w)], i1_v)
        pltpu.sync_copy(x_hbm.at[pl.ds(base, per_w)], buf_v)
        c0 = pltpu.async_copy(buf_v, xs_hbm.at[i0_v], sem)
        c1 = pltpu.async_copy(buf_v, xs_hbm.at[i1_v], sem)
        c0.wait()
        c1.wait()

    return d


@functools.lru_cache(maxsize=None)
def _make_sc_gather(n_idx, d):
    per_w = n_idx // _NW
    nch = per_w // _CH
    assert per_w % _CH == 0 and n_idx % _NW == 0

    @functools.partial(
        pl.kernel,
        mesh=_sc_mesh(),
        out_type=jax.ShapeDtypeStruct((n_idx, d), jnp.float32),
        scratch_types=[
            pltpu.VMEM((_CH,), jnp.int32),
            pltpu.VMEM((_CH, d), jnp.float32),
            pltpu.SemaphoreType.DMA,
        ],
    )
    def g(table_hbm, idx_hbm, out_hbm, idx_v, buf_v, sem):
        wid = lax.axis_index("s") * _NC + lax.axis_index("c")
        base = wid * per_w

        def chunk(i, carry):
            off = base + i * _CH
            pltpu.sync_copy(idx_hbm.at[pl.ds(off, _CH)], idx_v)
            pltpu.async_copy(table_hbm.at[idx_v], buf_v, sem).wait()
            pltpu.sync_copy(buf_v, out_hbm.at[pl.ds(off, _CH)])
            return carry

        lax.fori_loop(0, nch, chunk, 0)

    return g


def _ffn_body(texp_ref, nt_ref, x_ref, w1_ref, w3_ref, w2_ref, y_ref):
    t = pl.program_id(0)

    @pl.when(t < nt_ref[0])
    def _():
        x = x_ref[...]
        h1 = jnp.dot(x, w1_ref[0], preferred_element_type=jnp.float32)
        h3 = jnp.dot(x, w3_ref[0], preferred_element_type=jnp.float32)
        h = (h1 * jax.nn.sigmoid(h1)) * h3
        y_ref[...] = jnp.dot(h, w2_ref[0], preferred_element_type=jnp.float32)


def _ffn(texp, nt, xs, w1, w3, w2):
    grid_spec = pltpu.PrefetchScalarGridSpec(
        num_scalar_prefetch=2,
        grid=(_NT,),
        in_specs=[
            pl.BlockSpec((_BLK, _D),
                         lambda t, texp, nt: (jnp.minimum(t, nt[0] - 1), 0)),
            pl.BlockSpec((1, _D, _F), lambda t, texp, nt: (texp[t], 0, 0)),
            pl.BlockSpec((1, _D, _F), lambda t, texp, nt: (texp[t], 0, 0)),
            pl.BlockSpec((1, _F, _D), lambda t, texp, nt: (texp[t], 0, 0)),
        ],
        out_specs=pl.BlockSpec((_BLK, _D), lambda t, texp, nt: (t, 0)),
    )
    return pl.pallas_call(
        _ffn_body,
        grid_spec=grid_spec,
        out_shape=jax.ShapeDtypeStruct((_PAD, _D), jnp.float32),
    )(texp, nt, xs, w1, w3, w2)


def _add_body(z_ref, wa_ref, wb_ref, o_ref):
    z = z_ref[...]
    wa = wa_ref[0]
    wb = wb_ref[0]
    o_ref[...] = z[:, 0, :] * wa[:, None] + z[:, 1, :] * wb[:, None]


def _pair_add(z, wa2d, wb2d):
    return pl.pallas_call(
        _add_body,
        grid=(_T // _BLK,),
        in_specs=[
            pl.BlockSpec((_BLK, _K, _D), lambda t: (t, 0, 0)),
            pl.BlockSpec((8, _BLK), lambda t: (0, t)),
            pl.BlockSpec((8, _BLK), lambda t: (0, t)),
        ],
        out_specs=pl.BlockSpec((_BLK, _D), lambda t: (t, 0)),
        out_shape=jax.ShapeDtypeStruct((_T, _D), jnp.float32),
    )(z, wa2d, wb2d)


def kernel(hidden_states, gate_w, w1, w3, w2):
    orig_shape = hidden_states.shape
    x = hidden_states.reshape(_T, _D)
    e0_2d, e1_2d, p0_2d, p1_2d = _route(x, gate_w)
    pos0, pos1, texp, nt = _route_meta(e0_2d[0], e1_2d[0])
    xs = _make_sc_dispatch()(x, pos0, pos1)
    y = _ffn(texp, nt, xs, w1, w3, w2)
    pint = jnp.stack([pos0, pos1], axis=1).reshape(-1)
    z = _make_sc_gather(_A, _D)(y, pint)
    out = _pair_add(z.reshape(_T, _K, _D), p0_2d, p1_2d)
    return out.reshape(orig_shape)
